# baseline (speedup 1.0000x reference)
import functools

import jax
import jax.numpy as jnp
from jax import lax
from jax.experimental import pallas as pl
from jax.experimental.pallas import tpu as pltpu
from jax.experimental.pallas import tpu_sc as plsc

N_NODES = 100000
ROWS_N = 782
N_PAD = ROWS_N * 128
NCHUNK = N_PAD // 16

E_EDGES = 6400000
NC, NS, LANES = 2, 16, 16
ROW = 128
EROWS = E_EDGES // ROW
SB = 16
NSLOT = 3
NB = 98
R_TILE = NB * SB

CB = 368
NCB = 17


def _mesh():
    return plsc.VectorSubcoreMesh(
        core_axis_name="c", subcore_axis_name="s", num_cores=NC, num_subcores=NS
    )


def _rsqrt16(d):
    i = lax.bitcast_convert_type(d, jnp.int32)
    i = jnp.int32(0x5F3759DF) - lax.shift_right_logical(i, 1)
    y = lax.bitcast_convert_type(i, jnp.float32)
    for _ in range(3):
        y = y * (1.5 - 0.5 * d * y * y)
    return y


def _zero_acc(zbuf, acc, sid):
    for k in range(CB // 16):
        zbuf[pl.ds(k * 16, 16)] = jnp.zeros((16,), jnp.float32)

    def body(i, c):
        pltpu.sync_copy(zbuf, acc.at[pl.ds(sid * NCHUNK + i * CB, CB)])
        return c

    lax.fori_loop(0, NCB, body, 0)


def _writeout(zbuf, acc, out_hbm, cid, sid):
    def body(i, c):
        o = sid * NCHUNK + i * CB
        pltpu.sync_copy(acc.at[pl.ds(o, CB)], zbuf)
        pltpu.sync_copy(zbuf, out_hbm.at[pl.ds(cid * N_PAD + o, CB)])
        return c

    lax.fori_loop(0, NCB, body, 0)


def _edge_loop(ei_hbm, stage_s, stage_d, vals, table_v, acc, sem_st, sem_sc,
               wid, has_gather):
    base = wid * R_TILE
    nb_w = jnp.minimum(NB, (EROWS - base) // SB)

    def src_of(slot, j):
        return vals.at[slot, j] if has_gather else vals.at[0, 0]

    def start_stage(b, slot):
        @pl.when(b < nb_w)
        def _():
            r = base + b * SB
            if has_gather:
                pltpu.async_copy(
                    ei_hbm.at[pl.ds(r, SB)], stage_s.at[slot], sem_st)
            pltpu.async_copy(
                ei_hbm.at[pl.ds(EROWS + r, SB)], stage_d.at[slot], sem_st)

    def wait_stage(b, slot):
        @pl.when(b < nb_w)
        def _():
            if has_gather:
                pltpu.make_async_copy(
                    ei_hbm.at[pl.ds(0, SB)], stage_s.at[slot], sem_st).wait()
            pltpu.make_async_copy(
                ei_hbm.at[pl.ds(0, SB)], stage_d.at[slot], sem_st).wait()

    def fire(b, slot):
        @pl.when(b < nb_w)
        def _():
            for j in range(SB):
                pltpu.async_copy(
                    src_of(slot, j), acc.at[stage_d.at[slot, j]],
                    sem_sc, add=True)

    def drain(b, slot):
        @pl.when(b < nb_w)
        def _():
            for j in range(SB):
                pltpu.make_async_copy(
                    src_of(slot, j), acc.at[stage_d.at[slot, j]],
                    sem_sc).wait()

    def gath(slot):
        for j in range(SB):
            for k in range(ROW // LANES):
                idx = stage_s[slot, j, pl.ds(k * 16, 16)]
                vals[slot, j, pl.ds(k * 16, 16)] = plsc.load_gather(
                    table_v, [idx])

    def process(b, slot, drain_prev=True, lead=True):
        wait_stage(b, slot)
        if has_gather:
            gath(slot)
        fire(b, slot)
        if drain_prev:
            drain(b - 1, (slot - 1) % NSLOT)
        if lead:
            start_stage(b + 2, (slot + 2) % NSLOT)

    start_stage(0, 0)
    start_stage(1, 1)
    process(0, 0, drain_prev=False)
    process(1, 1)

    def tri(p, c):
        b = 3 * p + 2
        process(b, 2)
        process(b + 1, 0)
        process(b + 2, 1)
        return c

    lax.fori_loop(0, (NB - 2) // 3, tri, 0)
    drain(NB - 1, (NB - 1) % NSLOT)


def _deg_body(ei_hbm, out_hbm, stage_d, ones_v, zbuf, acc, sem_st, sem_sc):
    cid = lax.axis_index("c")
    sid = lax.axis_index("s")
    wid = cid * NS + sid
    _zero_acc(zbuf, acc, sid)
    for k in range(ROW // LANES):
        ones_v[0, 0, pl.ds(k * 16, 16)] = jnp.ones((16,), jnp.float32)
    plsc.subcore_barrier()
    _edge_loop(ei_hbm, None, stage_d, ones_v, None, acc, sem_st, sem_sc,
               wid, has_gather=False)
    plsc.subcore_barrier()
    _writeout(zbuf, acc, out_hbm, cid, sid)


_deg_call = functools.partial(
    pl.kernel,
    out_type=jax.ShapeDtypeStruct((NC * N_PAD,), jnp.float32),
    mesh=_mesh(),
    compiler_params=pltpu.CompilerParams(needs_layout_passes=False),
    scratch_types=[
        pltpu.VMEM((NSLOT, SB, ROW), jnp.int32),
        pltpu.VMEM((1, 1, ROW), jnp.float32),
        pltpu.VMEM((CB,), jnp.float32),
        pltpu.VMEM_SHARED((N_PAD,), jnp.float32),
        pltpu.SemaphoreType.DMA,
        pltpu.SemaphoreType.DMA,
    ],
)(_deg_body)


def _agg1_body(ei_hbm, degp_hbm, x_hbm, out_hbm, g1_hbm,
               stage_s, stage_d, vals, bufa, bufb, bufc, zbuf,
               table_v, acc, sem_st, sem_sc):
    cid = lax.axis_index("c")
    sid = lax.axis_index("s")
    wid = cid * NS + sid
    _zero_acc(zbuf, acc, sid)
    def bchunk(i, c):
        o = sid * NCHUNK + i * CB
        pltpu.async_copy(degp_hbm.at[pl.ds(o, CB)], bufa, sem_st)
        pltpu.async_copy(degp_hbm.at[pl.ds(N_PAD + o, CB)], bufb, sem_st)
        pltpu.async_copy(x_hbm.at[pl.ds(o, CB)], bufc, sem_st)
        pltpu.make_async_copy(degp_hbm.at[pl.ds(o, CB)], bufa, sem_st).wait()
        pltpu.make_async_copy(degp_hbm.at[pl.ds(o, CB)], bufb, sem_st).wait()
        pltpu.make_async_copy(x_hbm.at[pl.ds(o, CB)], bufc, sem_st).wait()

        def vec(k, c2):
            ds = pl.ds(k * 16, 16)
            d = bufa[ds] + bufb[ds] + 1.0
            y = _rsqrt16(d)
            bufa[ds] = y * bufc[ds]
            return c2

        lax.fori_loop(0, CB // 16, vec, 0)
        pltpu.sync_copy(bufa, g1_hbm.at[pl.ds(cid * N_PAD + o, CB)])
        return c

    lax.fori_loop(0, NCB, bchunk, 0)
    plsc.subcore_barrier()
    pltpu.sync_copy(g1_hbm.at[pl.ds(cid * N_PAD, N_PAD)], table_v)
    _edge_loop(ei_hbm, stage_s, stage_d, vals, table_v, acc, sem_st, sem_sc,
               wid, has_gather=True)
    plsc.subcore_barrier()
    _writeout(zbuf, acc, out_hbm, cid, sid)


_agg1_call = functools.partial(
    pl.kernel,
    out_type=[
        jax.ShapeDtypeStruct((NC * N_PAD,), jnp.float32),
        jax.ShapeDtypeStruct((NC * N_PAD,), jnp.float32),
    ],
    mesh=_mesh(),
    compiler_params=pltpu.CompilerParams(needs_layout_passes=False),
    scratch_types=[
        pltpu.VMEM((NSLOT, SB, ROW), jnp.int32),
        pltpu.VMEM((NSLOT, SB, ROW), jnp.int32),
        pltpu.VMEM((NSLOT, SB, ROW), jnp.float32),
        pltpu.VMEM((CB,), jnp.float32),
        pltpu.VMEM((CB,), jnp.float32),
        pltpu.VMEM((CB,), jnp.float32),
        pltpu.VMEM((CB,), jnp.float32),
        pltpu.VMEM((N_PAD,), jnp.float32),
        pltpu.VMEM_SHARED((N_PAD,), jnp.float32),
        pltpu.SemaphoreType.DMA,
        pltpu.SemaphoreType.DMA,
    ],
)(_agg1_body)


def _agg2_body(ei_hbm, degp_hbm, r1p_hbm, x_hbm, w_hbm, out_hbm, h2_hbm,
               g2_hbm, stage_s, stage_d, vals, w_v, bufa, bufb, bufc, bufd,
               bufe, zbuf, table_v, acc, sem_st, sem_sc):
    cid = lax.axis_index("c")
    sid = lax.axis_index("s")
    wid = cid * NS + sid
    _zero_acc(zbuf, acc, sid)
    pltpu.sync_copy(w_hbm, w_v)
    def bchunk(i, c):
        o = sid * NCHUNK + i * CB
        pltpu.async_copy(degp_hbm.at[pl.ds(o, CB)], bufa, sem_st)
        pltpu.async_copy(degp_hbm.at[pl.ds(N_PAD + o, CB)], bufb, sem_st)
        pltpu.async_copy(r1p_hbm.at[pl.ds(o, CB)], bufc, sem_st)
        pltpu.async_copy(r1p_hbm.at[pl.ds(N_PAD + o, CB)], bufd, sem_st)
        pltpu.async_copy(x_hbm.at[pl.ds(o, CB)], bufe, sem_st)
        for buf in (bufa, bufb, bufc, bufd, bufe):
            pltpu.make_async_copy(
                degp_hbm.at[pl.ds(o, CB)], buf, sem_st).wait()

        def vec(k, c2):
            ds = pl.ds(k * 16, 16)
            d = bufa[ds] + bufb[ds] + 1.0
            y = _rsqrt16(d)
            agg1 = y * (bufc[ds] + bufd[ds]) + y * y * bufe[ds]
            h = jnp.zeros((16,), jnp.float32)
            for j in range(8):
                h = h + jnp.maximum(
                    agg1 * w_v[j, pl.ds(0, 16)] + w_v[8 + j, pl.ds(0, 16)],
                    0.0) * w_v[16 + j, pl.ds(0, 16)]
            bufa[ds] = h
            bufb[ds] = y * h
            return c2

        lax.fori_loop(0, CB // 16, vec, 0)

        @pl.when(cid == 0)
        def _():
            pltpu.sync_copy(bufa, h2_hbm.at[pl.ds(o, CB)])
        pltpu.sync_copy(bufb, g2_hbm.at[pl.ds(cid * N_PAD + o, CB)])
        return c

    lax.fori_loop(0, NCB, bchunk, 0)
    plsc.subcore_barrier()
    pltpu.sync_copy(g2_hbm.at[pl.ds(cid * N_PAD, N_PAD)], table_v)
    _edge_loop(ei_hbm, stage_s, stage_d, vals, table_v, acc, sem_st, sem_sc,
               wid, has_gather=True)
    plsc.subcore_barrier()
    _writeout(zbuf, acc, out_hbm, cid, sid)


_agg2_call = functools.partial(
    pl.kernel,
    out_type=[
        jax.ShapeDtypeStruct((NC * N_PAD,), jnp.float32),
        jax.ShapeDtypeStruct((N_PAD,), jnp.float32),
        jax.ShapeDtypeStruct((NC * N_PAD,), jnp.float32),
    ],
    mesh=_mesh(),
    compiler_params=pltpu.CompilerParams(needs_layout_passes=False),
    scratch_types=[
        pltpu.VMEM((NSLOT, SB, ROW), jnp.int32),
        pltpu.VMEM((NSLOT, SB, ROW), jnp.int32),
        pltpu.VMEM((NSLOT, SB, ROW), jnp.float32),
        pltpu.VMEM((24, 16), jnp.float32),
        pltpu.VMEM((CB,), jnp.float32),
        pltpu.VMEM((CB,), jnp.float32),
        pltpu.VMEM((CB,), jnp.float32),
        pltpu.VMEM((CB,), jnp.float32),
        pltpu.VMEM((CB,), jnp.float32),
        pltpu.VMEM((CB,), jnp.float32),
        pltpu.VMEM((N_PAD,), jnp.float32),
        pltpu.VMEM_SHARED((N_PAD,), jnp.float32),
        pltpu.SemaphoreType.DMA,
        pltpu.SemaphoreType.DMA,
    ],
)(_agg2_body)


def _ew3_body(degp, rawp, h2, b2, out_o):
    deg = degp[0] + degp[1] + 1.0
    dv = lax.rsqrt(deg)
    out_o[...] = dv * (rawp[0] + rawp[1]) + dv * dv * h2[...] + b2[0]


def _ew3(degp, rawp, h2, b2):
    return pl.pallas_call(
        _ew3_body,
        in_specs=[pl.BlockSpec(), pl.BlockSpec(), pl.BlockSpec(),
                  pl.BlockSpec(memory_space=pltpu.SMEM)],
        out_shape=jax.ShapeDtypeStruct((ROWS_N, 128), jnp.float32),
    )(degp, rawp, h2, b2)


def kernel(x, edge_index, W1, b1, W2, b2):
    xpad = jnp.pad(x[:, 0], (0, N_PAD - N_NODES))
    ei2 = edge_index.reshape(2 * EROWS, ROW)
    wmat = jnp.concatenate([
        jnp.broadcast_to(W1.reshape(8, 1), (8, 16)),
        jnp.broadcast_to(b1.reshape(8, 1), (8, 16)),
        jnp.broadcast_to(W2.reshape(8, 1), (8, 16)),
    ], axis=0)

    degp = _deg_call(ei2)
    raw1p, _ = _agg1_call(ei2, degp, xpad)
    raw2p, h2, _ = _agg2_call(ei2, degp, raw1p, xpad, wmat)
    out = _ew3(degp.reshape(NC, ROWS_N, 128), raw2p.reshape(NC, ROWS_N, 128),
               h2.reshape(ROWS_N, 128), b2)
    return out.reshape(N_PAD)[:N_NODES][:, None]


# speedup vs baseline: 1.1230x; 1.1230x over previous
import functools

import jax
import jax.numpy as jnp
from jax import lax
from jax.experimental import pallas as pl
from jax.experimental.pallas import tpu as pltpu
from jax.experimental.pallas import tpu_sc as plsc

N_NODES = 100000
ROWS_N = 782
N_PAD = ROWS_N * 128
NCHUNK = N_PAD // 16

E_EDGES = 6400000
NC, NS, LANES = 2, 16, 16
ROW = 128
EROWS = E_EDGES // ROW
SB = 16
NSLOT = 3
NB = 98
R_TILE = NB * SB


def _mesh():
    return plsc.VectorSubcoreMesh(
        core_axis_name="c", subcore_axis_name="s", num_cores=NC, num_subcores=NS
    )


def _zero_acc(zbuf, acc, sid):
    def zf(j, c):
        zbuf[pl.ds(j * 16, 16)] = jnp.zeros((16,), jnp.float32)
        return c

    lax.fori_loop(0, NCHUNK // 16, zf, 0)
    pltpu.sync_copy(zbuf, acc.at[pl.ds(sid * NCHUNK, NCHUNK)])


def _writeout(zbuf, acc, out_hbm, cid, sid):
    pltpu.sync_copy(acc.at[pl.ds(sid * NCHUNK, NCHUNK)], zbuf)
    pltpu.sync_copy(zbuf, out_hbm.at[pl.ds(cid * N_PAD + sid * NCHUNK, NCHUNK)])


def _make_edge_body(has_gather):
    def body(*refs):
        if has_gather:
            (ei_hbm, tab_hbm, out_hbm,
             stage_s, stage_d, vals, table_v, zbuf, acc, sem_st, sem_sc) = refs
        else:
            (ei_hbm, out_hbm, stage_d, ones_v, zbuf, acc, sem_st, sem_sc) = refs
        cid = lax.axis_index("c")
        sid = lax.axis_index("s")
        wid = cid * NS + sid
        _zero_acc(zbuf, acc, sid)
        if has_gather:
            pltpu.sync_copy(tab_hbm, table_v)
        else:
            for k in range(ROW // LANES):
                ones_v[pl.ds(k * 16, 16)] = jnp.ones((16,), jnp.float32)
        plsc.subcore_barrier()
        base = wid * R_TILE
        nb_w = jnp.minimum(NB, (EROWS - base) // SB)

        def src_of(slot, j):
            return vals.at[slot, j] if has_gather else ones_v

        def start_stage(b, slot):
            @pl.when(b < nb_w)
            def _():
                r = base + b * SB
                if has_gather:
                    pltpu.async_copy(
                        ei_hbm.at[pl.ds(r, SB)], stage_s.at[slot], sem_st)
                pltpu.async_copy(
                    ei_hbm.at[pl.ds(EROWS + r, SB)], stage_d.at[slot], sem_st)

        def wait_stage(b, slot):
            @pl.when(b < nb_w)
            def _():
                if has_gather:
                    pltpu.make_async_copy(
                        ei_hbm.at[pl.ds(0, SB)], stage_s.at[slot], sem_st).wait()
                pltpu.make_async_copy(
                    ei_hbm.at[pl.ds(0, SB)], stage_d.at[slot], sem_st).wait()

        def fire(b, slot):
            @pl.when(b < nb_w)
            def _():
                for j in range(SB):
                    pltpu.async_copy(
                        src_of(slot, j), acc.at[stage_d.at[slot, j]],
                        sem_sc, add=True)

        def drain(b, slot):
            @pl.when(b < nb_w)
            def _():
                for j in range(SB):
                    pltpu.make_async_copy(
                        src_of(slot, j), acc.at[stage_d.at[slot, j]],
                        sem_sc).wait()

        def gath(slot):
            for j in range(SB):
                for k in range(ROW // LANES):
                    idx = stage_s[slot, j, pl.ds(k * 16, 16)]
                    vals[slot, j, pl.ds(k * 16, 16)] = plsc.load_gather(
                        table_v, [idx])

        def process(b, slot, drain_prev=True, lead=True):
            wait_stage(b, slot)
            if has_gather:
                gath(slot)
            fire(b, slot)
            if drain_prev:
                drain(b - 1, (slot - 1) % NSLOT)
            if lead:
                start_stage(b + 2, (slot + 2) % NSLOT)

        start_stage(0, 0)
        start_stage(1, 1)
        process(0, 0, drain_prev=False)
        process(1, 1)

        def tri(p, c):
            b = 3 * p + 2
            process(b, 2)
            process(b + 1, 0)
            process(b + 2, 1)
            return c

        lax.fori_loop(0, (NB - 2) // 3, tri, 0)
        drain(NB - 1, (NB - 1) % NSLOT)

        plsc.subcore_barrier()
        _writeout(zbuf, acc, out_hbm, cid, sid)

    return body


_deg_call = functools.partial(
    pl.kernel,
    out_type=jax.ShapeDtypeStruct((NC * N_PAD,), jnp.float32),
    mesh=_mesh(),
    compiler_params=pltpu.CompilerParams(needs_layout_passes=False),
    scratch_types=[
        pltpu.VMEM((NSLOT, SB, ROW), jnp.int32),
        pltpu.VMEM((ROW,), jnp.float32),
        pltpu.VMEM((NCHUNK,), jnp.float32),
        pltpu.VMEM_SHARED((N_PAD,), jnp.float32),
        pltpu.SemaphoreType.DMA,
        pltpu.SemaphoreType.DMA,
    ],
)(_make_edge_body(False))


_agg_call = functools.partial(
    pl.kernel,
    out_type=jax.ShapeDtypeStruct((NC * N_PAD,), jnp.float32),
    mesh=_mesh(),
    compiler_params=pltpu.CompilerParams(needs_layout_passes=False),
    scratch_types=[
        pltpu.VMEM((NSLOT, SB, ROW), jnp.int32),
        pltpu.VMEM((NSLOT, SB, ROW), jnp.int32),
        pltpu.VMEM((NSLOT, SB, ROW), jnp.float32),
        pltpu.VMEM((N_PAD,), jnp.float32),
        pltpu.VMEM((NCHUNK,), jnp.float32),
        pltpu.VMEM_SHARED((N_PAD,), jnp.float32),
        pltpu.SemaphoreType.DMA,
        pltpu.SemaphoreType.DMA,
    ],
)(_make_edge_body(True))


def _ew1_body(degp, xp, dinv_o, g1_o):
    deg = degp[0] + degp[1] + 1.0
    dinv = lax.rsqrt(deg)
    dinv_o[...] = dinv
    g1_o[...] = dinv * xp[...]


def _ew2_body(rawp, dinv, xp, w1, b1, w2, h2_o, g2_o):
    dv = dinv[...]
    agg1 = dv * (rawp[0] + rawp[1]) + dv * dv * xp[...]
    acc = jnp.zeros_like(agg1)
    for j in range(8):
        acc = acc + jnp.maximum(agg1 * w1[0, j] + b1[j], 0.0) * w2[j, 0]
    h2_o[...] = acc
    g2_o[...] = dv * acc


def _ew3_body(rawp, dinv, h2, b2, out_o):
    dv = dinv[...]
    out_o[...] = dv * (rawp[0] + rawp[1]) + dv * dv * h2[...] + b2[0]


_SMEM = pl.BlockSpec(memory_space=pltpu.SMEM)


def _ew1(degp, xp):
    return pl.pallas_call(
        _ew1_body,
        out_shape=[
            jax.ShapeDtypeStruct((ROWS_N, 128), jnp.float32),
            jax.ShapeDtypeStruct((ROWS_N, 128), jnp.float32),
        ],
    )(degp, xp)


def _ew2(rawp, dinv, xp, w1, b1, w2):
    return pl.pallas_call(
        _ew2_body,
        in_specs=[pl.BlockSpec(), pl.BlockSpec(), pl.BlockSpec(), _SMEM, _SMEM, _SMEM],
        out_shape=[
            jax.ShapeDtypeStruct((ROWS_N, 128), jnp.float32),
            jax.ShapeDtypeStruct((ROWS_N, 128), jnp.float32),
        ],
    )(rawp, dinv, xp, w1, b1, w2)


def _ew3(rawp, dinv, h2, b2):
    return pl.pallas_call(
        _ew3_body,
        in_specs=[pl.BlockSpec(), pl.BlockSpec(), pl.BlockSpec(), _SMEM],
        out_shape=jax.ShapeDtypeStruct((ROWS_N, 128), jnp.float32),
    )(rawp, dinv, h2, b2)


def kernel(x, edge_index, W1, b1, W2, b2):
    xf = x[:, 0]
    xp = jnp.pad(xf, (0, N_PAD - N_NODES)).reshape(ROWS_N, 128)
    ei2 = edge_index.reshape(2 * EROWS, ROW)

    degp = _deg_call(ei2)
    dinv, g1 = _ew1(degp.reshape(NC, ROWS_N, 128), xp)
    raw1p = _agg_call(ei2, g1.reshape(N_PAD))
    h2, g2 = _ew2(raw1p.reshape(NC, ROWS_N, 128), dinv, xp, W1, b1, W2)
    raw2p = _agg_call(ei2, g2.reshape(N_PAD))
    out = _ew3(raw2p.reshape(NC, ROWS_N, 128), dinv, h2, b2)
    return out.reshape(N_PAD)[:N_NODES][:, None]
